# baseline (speedup 1.0000x reference)
import functools

import jax
import jax.numpy as jnp
from jax import lax
from jax.experimental import pallas as pl
from jax.experimental.pallas import tpu as pltpu
from jax.experimental.pallas import tpu_sc as plsc

VOCAB = 100000
DESC = 768
PROJ = 128
BATCH = 16384


@functools.cache
def _make_sc_gather(B, D):
    NC, NS = 2, 16
    NW = NC * NS
    b_per_w = B // NW
    C = 32
    n_chunks = b_per_w // C
    D2 = D // 2
    mesh = plsc.VectorSubcoreMesh(core_axis_name="c", subcore_axis_name="s")

    @functools.partial(
        pl.kernel,
        mesh=mesh,
        compiler_params=pltpu.CompilerParams(
            needs_layout_passes=False, use_tc_tiling_on_sc=False),
        out_type=jax.ShapeDtypeStruct((B, D), jnp.bfloat16),
        scratch_types=[
            pltpu.VMEM((b_per_w,), jnp.int32),
            pltpu.VMEM((C, D), jnp.float32),
            pltpu.VMEM((C, D), jnp.float32),
            pltpu.VMEM((C, D), jnp.bfloat16),
            pltpu.VMEM((C, D), jnp.bfloat16),
            pltpu.SemaphoreType.DMA,
            pltpu.SemaphoreType.DMA,
            pltpu.SemaphoreType.DMA,
            pltpu.SemaphoreType.DMA,
        ],
    )
    def gather(idx_hbm, table_hbm, out_hbm, idx_v, f0, f1, p0, p1,
               g0, g1, o0, o1):
        wid = lax.axis_index("s") * NC + lax.axis_index("c")
        base = wid * b_per_w
        pltpu.sync_copy(idx_hbm.at[pl.ds(base, b_per_w)], idx_v)

        def convert(fbuf, pbuf):
            def row(r, carry):
                for j in range(D // 32):
                    va = fbuf[r, pl.ds(j * 32, 16)]
                    vb = fbuf[r, pl.ds(j * 32 + 16, 16)]
                    pbuf[r, pl.ds(j * 32, 32)] = plsc.pack(
                        va, vb, format=plsc.PackFormat.INTERLEAVED)
                return carry
            lax.fori_loop(0, C, row, 0)

        pltpu.async_copy(table_hbm.at[idx_v.at[pl.ds(0, C)]], f0, g0)

        def chunk(i, carry):
            def do(fb, pb, gs, osem, fb_n, gs_n):
                @pl.when(i + 1 < n_chunks)
                def _():
                    off = pl.multiple_of((i + 1) * C, 8)
                    pltpu.async_copy(
                        table_hbm.at[idx_v.at[pl.ds(off, C)]], fb_n, gs_n)
                pltpu.make_async_copy(
                    table_hbm.at[idx_v.at[pl.ds(0, C)]], fb, gs).wait()
                @pl.when(i >= 2)
                def _():
                    pltpu.make_async_copy(
                        pb, out_hbm.at[pl.ds(base, C)], osem).wait()
                convert(fb, pb)
                pltpu.async_copy(
                    pb, out_hbm.at[pl.ds(base + i * C, C)], osem)

            @pl.when(lax.rem(i, 2) == 0)
            def _():
                do(f0, p0, g0, o0, f1, g1)

            @pl.when(lax.rem(i, 2) == 1)
            def _():
                do(f1, p1, g1, o1, f0, g0)

            return carry

        lax.fori_loop(0, n_chunks, chunk, 0)
        pltpu.make_async_copy(p0, out_hbm.at[pl.ds(base, C)], o0).wait()
        pltpu.make_async_copy(p1, out_hbm.at[pl.ds(base, C)], o1).wait()

    return gather


_BM = 2048


def _proj_body(vals_ref, emb_ref, wp1_ref, wp2p_ref, wvalt_ref, bval_ref,
               bproj_ref, out_ref):
    c1 = lax.dot_general(wvalt_ref[...], wp1_ref[...], (((1,), (1,)), ((), ())),
                         preferred_element_type=jnp.float32)
    c0 = lax.dot_general(bval_ref[...], wp1_ref[...], (((1,), (1,)), ((), ())),
                         preferred_element_type=jnp.float32) + bproj_ref[...]
    emb = emb_ref[...].astype(jnp.float32)
    emb_term = lax.dot_general(emb, wp2p_ref[...], (((1,), (1,)), ((), ())),
                               preferred_element_type=jnp.float32)
    val_term = lax.dot_general(vals_ref[...], c1, (((1,), (0,)), ((), ())),
                               preferred_element_type=jnp.float32)
    out_ref[...] = emb_term + val_term + c0


def _proj(values, emb_bf16, wp1, wp2p, W_val_t, b_val2, b_proj2):
    grid = (BATCH // _BM,)
    return pl.pallas_call(
        _proj_body,
        grid=grid,
        in_specs=[
            pl.BlockSpec((_BM, 1), lambda i: (i, 0)),
            pl.BlockSpec((_BM, DESC), lambda i: (i, 0)),
            pl.BlockSpec((PROJ, DESC), lambda i: (0, 0)),
            pl.BlockSpec((PROJ, DESC), lambda i: (0, 0)),
            pl.BlockSpec((1, DESC), lambda i: (0, 0)),
            pl.BlockSpec((1, DESC), lambda i: (0, 0)),
            pl.BlockSpec((1, PROJ), lambda i: (0, 0)),
        ],
        out_specs=pl.BlockSpec((_BM, PROJ), lambda i: (i, 0)),
        out_shape=jax.ShapeDtypeStruct((BATCH, PROJ), jnp.float32),
    )(values, emb_bf16, wp1, wp2p, W_val_t, b_val2, b_proj2)


def kernel(description_idx, values, embedded_descriptions, W_val, b_val, W_proj, b_proj):
    idx = description_idx.astype(jnp.int32)
    emb_bf16 = _make_sc_gather(BATCH, DESC)(idx, embedded_descriptions)
    wp1 = W_proj[:, :DESC]
    wp2 = W_proj[:, DESC:]
    wp2p = wp2.reshape(PROJ, DESC // 32, 2, 16).transpose(0, 1, 3, 2).reshape(PROJ, DESC)
    return _proj(
        values,
        emb_bf16,
        wp1,
        wp2p,
        W_val.reshape(1, DESC),
        b_val.reshape(1, DESC),
        b_proj.reshape(1, PROJ),
    )


# speedup vs baseline: 6.2428x; 6.2428x over previous
import functools

import jax
import jax.numpy as jnp
from jax import lax
from jax.experimental import pallas as pl
from jax.experimental.pallas import tpu as pltpu
from jax.experimental.pallas import tpu_sc as plsc

VOCAB = 100000
DESC = 768
PROJ = 128
BATCH = 16384


@functools.cache
def _make_sc_gather(B, D):
    NC, NS = 2, 16
    NW = NC * NS
    b_per_w = B // NW
    C = 64
    n_chunks = b_per_w // C
    mesh = plsc.VectorSubcoreMesh(core_axis_name="c", subcore_axis_name="s")

    @functools.partial(
        pl.kernel,
        mesh=mesh,
        out_type=jax.ShapeDtypeStruct((B, D), jnp.float32),
        scratch_types=[
            pltpu.VMEM((b_per_w,), jnp.int32),
            pltpu.VMEM((C, D), jnp.float32),
            pltpu.VMEM((C, D), jnp.float32),
            pltpu.SemaphoreType.DMA,
            pltpu.SemaphoreType.DMA,
            pltpu.SemaphoreType.DMA,
            pltpu.SemaphoreType.DMA,
        ],
    )
    def gather(idx_hbm, table_hbm, out_hbm, idx_v, f0, f1, g0, g1, o0, o1):
        wid = lax.axis_index("s") * NC + lax.axis_index("c")
        base = wid * b_per_w
        pltpu.sync_copy(idx_hbm.at[pl.ds(base, b_per_w)], idx_v)

        pltpu.async_copy(table_hbm.at[idx_v.at[pl.ds(0, C)]], f0, g0)

        def chunk(i, carry):
            def do(fb, gs, osem, fb_n, gs_n):
                @pl.when(i + 1 < n_chunks)
                def _():
                    off = pl.multiple_of((i + 1) * C, 8)
                    pltpu.async_copy(
                        table_hbm.at[idx_v.at[pl.ds(off, C)]], fb_n, gs_n)
                pltpu.make_async_copy(
                    table_hbm.at[idx_v.at[pl.ds(0, C)]], fb, gs).wait()
                @pl.when(i >= 2)
                def _():
                    pltpu.make_async_copy(
                        fb, out_hbm.at[pl.ds(base, C)], osem).wait()
                pltpu.async_copy(
                    fb, out_hbm.at[pl.ds(base + i * C, C)], osem)

            @pl.when(lax.rem(i, 2) == 0)
            def _():
                do(f0, g0, o0, f1, g1)

            @pl.when(lax.rem(i, 2) == 1)
            def _():
                do(f1, g1, o1, f0, g0)

            return carry

        lax.fori_loop(0, n_chunks, chunk, 0)
        pltpu.make_async_copy(f0, out_hbm.at[pl.ds(base, C)], o0).wait()
        pltpu.make_async_copy(f1, out_hbm.at[pl.ds(base, C)], o1).wait()

    return gather


_BM = 2048


def _proj_body(vals_ref, emb_ref, wproj_ref, wvalt_ref, bval_ref, bproj_ref, out_ref):
    wp1 = wproj_ref[:, :DESC]
    wp2 = wproj_ref[:, DESC:]
    c1 = lax.dot_general(wvalt_ref[...], wp1, (((1,), (1,)), ((), ())),
                         preferred_element_type=jnp.float32)
    c0 = lax.dot_general(bval_ref[...], wp1, (((1,), (1,)), ((), ())),
                         preferred_element_type=jnp.float32) + bproj_ref[...]
    emb_term = lax.dot_general(emb_ref[...], wp2, (((1,), (1,)), ((), ())),
                               preferred_element_type=jnp.float32)
    val_term = lax.dot_general(vals_ref[...], c1, (((1,), (0,)), ((), ())),
                               preferred_element_type=jnp.float32)
    out_ref[...] = emb_term + val_term + c0


def _proj(values, emb, W_proj, W_val_t, b_val2, b_proj2):
    n = values.shape[0]
    grid = (n // _BM,)
    return pl.pallas_call(
        _proj_body,
        grid=grid,
        in_specs=[
            pl.BlockSpec((_BM, 1), lambda i: (i, 0)),
            pl.BlockSpec((_BM, DESC), lambda i: (i, 0)),
            pl.BlockSpec((PROJ, 2 * DESC), lambda i: (0, 0)),
            pl.BlockSpec((1, DESC), lambda i: (0, 0)),
            pl.BlockSpec((1, DESC), lambda i: (0, 0)),
            pl.BlockSpec((1, PROJ), lambda i: (0, 0)),
        ],
        out_specs=pl.BlockSpec((_BM, PROJ), lambda i: (i, 0)),
        out_shape=jax.ShapeDtypeStruct((n, PROJ), jnp.float32),
    )(values, emb, W_proj, W_val_t, b_val2, b_proj2)


def kernel(description_idx, values, embedded_descriptions, W_val, b_val, W_proj, b_proj):
    idx = description_idx.astype(jnp.int32)
    emb = _make_sc_gather(BATCH, DESC)(idx, embedded_descriptions)
    return _proj(
        values,
        emb,
        W_proj,
        W_val.reshape(1, DESC),
        b_val.reshape(1, DESC),
        b_proj.reshape(1, PROJ),
    )
